# baseline (speedup 1.0000x reference)
import jax
import jax.numpy as jnp
from jax import lax
from jax.experimental import pallas as pl
from jax.experimental.pallas import tpu as pltpu
from jax.experimental.pallas import tpu_sc as plsc

N = 50000
E = 800000
F = 100
Q = 4
QC = 32
CNT_Q = 3
CNT_LANE = 4

ROWS = 2000

NCORES = 2
NTILES = 16
EPT = E // NTILES
CB = 2000
CBP = 2048
NB = EPT // CB
G = 128
NGRP = CBP // G
ACC_R = 50048
DUMMY0 = N
ZSHARE = ACC_R // NTILES
ZROWS = 136
CO_SHARE = 3128
CO_LAST = N - 15 * CO_SHARE


def _matmul_kernel(x_ref, w_ref, o_ref):
    o_ref[...] = jnp.dot(x_ref[...], w_ref[0],
                         preferred_element_type=jnp.float32)


def _ggc_matmul(x, w):
    grid = (Q, N // ROWS)
    return pl.pallas_call(
        _matmul_kernel,
        grid=grid,
        in_specs=[
            pl.BlockSpec((ROWS, F), lambda q, i: (i, 0)),
            pl.BlockSpec((1, F, QC), lambda q, i: (q, 0, 0)),
        ],
        out_specs=pl.BlockSpec((ROWS, QC),
                               lambda q, i: (q * (N // ROWS) + i, 0)),
        out_shape=jax.ShapeDtypeStruct((Q * N, QC), jnp.float32),
    )(x, w)


def _sc_body(m_hbm, ei_hbm, w_hbm, out_hbm,
             src_v, dst_v, w_v, src2d, dst2d,
             rows_g0, rows_g1, rows_s0, rows_s1, acc,
             gs0, gs1, ss0, ss1, st):
    cid = lax.axis_index("c")
    tid = lax.axis_index("s")
    lane = lax.iota(jnp.int32, 16)
    zeros16 = jnp.zeros((16,), jnp.float32)

    for p in range(Q // NCORES):
        q = cid + NCORES * p
        qn = q * N

        for zr in range(G):
            rows_s0[zr, pl.ds(0, 16)] = zeros16
            rows_s0[zr, pl.ds(16, 16)] = zeros16

        def zero_step(j, _):
            pltpu.sync_copy(rows_s0, acc.at[pl.ds(tid * ZSHARE + j * G, G)])
            return 0
        lax.fori_loop(0, ZSHARE // G, zero_step, 0)
        pltpu.sync_copy(rows_s0.at[pl.ds(0, ZSHARE - (ZSHARE // G) * G)],
                        acc.at[pl.ds(tid * ZSHARE + (ZSHARE // G) * G,
                                     ZSHARE - (ZSHARE // G) * G)])

        dummy_src = qn + tid * 16 + lane
        for k in range(5, 8):
            src2d[15, pl.ds(k * 16, 16)] = dummy_src
            dst2d[15, pl.ds(k * 16, 16)] = DUMMY0 + lane
        for k in range(CB // 16, CBP // 16):
            w_v[pl.ds(k * 16, 16)] = zeros16
            w_v[pl.ds(CBP + k * 16, 16)] = zeros16

        def issue_stage(bk, whalf):
            ebase = tid * EPT + bk * CB
            pltpu.async_copy(ei_hbm.at[0, pl.ds(ebase, CB)], src_v, st)
            pltpu.async_copy(ei_hbm.at[1, pl.ds(ebase, CB)], dst_v, st)
            pltpu.async_copy(w_hbm.at[pl.ds(ebase, CB)],
                             w_v.at[pl.ds(whalf, CB)], st)

        issue_stage(0, 0)
        plsc.subcore_barrier()

        def block_step(bk, _):
            @pl.when(bk > 0)
            def _():
                pltpu.make_async_copy(rows_s0, acc.at[dst2d.at[0]],
                                      ss0).wait()
                pltpu.make_async_copy(rows_s1, acc.at[dst2d.at[1]],
                                      ss1).wait()

            half = lax.rem(bk, 2) * CBP
            ebase = tid * EPT + bk * CB
            pltpu.make_async_copy(ei_hbm.at[0, pl.ds(ebase, CB)],
                                  src_v, st).wait()
            pltpu.make_async_copy(ei_hbm.at[1, pl.ds(ebase, CB)],
                                  dst_v, st).wait()
            pltpu.make_async_copy(w_hbm.at[pl.ds(ebase, CB)],
                                  w_v.at[pl.ds(half, CB)], st).wait()

            for k in range(CB // 16):
                r, c = k // 8, (k % 8) * 16
                src2d[r, pl.ds(c, 16)] = src_v[pl.ds(k * 16, 16)] + qn
                dst2d[r, pl.ds(c, 16)] = dst_v[pl.ds(k * 16, 16)]

            @pl.when(bk < NB - 1)
            def _():
                issue_stage(bk + 1, CBP - half)

            indf = jnp.minimum(jnp.abs(lane - CNT_LANE), 1).astype(jnp.float32)
            cnt_add = cid.astype(jnp.float32) * (1.0 - indf)
            cnt_keep = 1.0 - cnt_add

            pltpu.async_copy(m_hbm.at[src2d.at[0]], rows_g0, gs0)
            pltpu.async_copy(m_hbm.at[src2d.at[1]], rows_g1, gs1)

            bufs = ((rows_g0, rows_s0, gs0, ss0), (rows_g1, rows_s1, gs1, ss1))

            def gpair(i, _):
                for bi, (rg, rs, gs, ss) in enumerate(bufs):
                    g = 2 * i + bi
                    pltpu.make_async_copy(m_hbm.at[src2d.at[g]], rg,
                                          gs).wait()

                    @pl.when(i >= 1)
                    def _():
                        pltpu.make_async_copy(rs, acc.at[dst2d.at[g]],
                                              ss).wait()

                    base = half + g * G
                    for j16 in range(G // 16):
                        w16 = w_v[pl.ds(base + j16 * 16, 16)]
                        for l in range(16):
                            j = j16 * 16 + l
                            wv = w16.at[jnp.full((16,), l, jnp.int32)].get(
                                mode="promise_in_bounds")
                            v0 = rg[j, pl.ds(0, 16)] * wv
                            if p == 1:
                                v0 = v0 * cnt_keep + cnt_add
                            rs[j, pl.ds(0, 16)] = v0
                            rs[j, pl.ds(16, 16)] = rg[j, pl.ds(16, 16)] * wv
                    pltpu.async_copy(rs, acc.at[dst2d.at[g]], ss, add=True)

                    @pl.when(i < NGRP // 2 - 1)
                    def _():
                        pltpu.async_copy(m_hbm.at[src2d.at[g + 2]], rg, gs)
                return 0

            lax.fori_loop(0, NGRP // 2, gpair, 0, unroll=2)
            return 0

        lax.fori_loop(0, NB, block_step, 0)
        pltpu.make_async_copy(rows_s0, acc.at[dst2d.at[0]], ss0).wait()
        pltpu.make_async_copy(rows_s1, acc.at[dst2d.at[1]], ss1).wait()
        plsc.subcore_barrier()

        @pl.when(tid < NTILES - 1)
        def _():
            pltpu.sync_copy(acc.at[pl.ds(tid * CO_SHARE, CO_SHARE)],
                            out_hbm.at[pl.ds(qn + tid * CO_SHARE, CO_SHARE)])

        @pl.when(tid == NTILES - 1)
        def _():
            pltpu.sync_copy(
                acc.at[pl.ds(15 * CO_SHARE, CO_LAST)],
                out_hbm.at[pl.ds(qn + 15 * CO_SHARE, CO_LAST)])
        plsc.subcore_barrier()


def _sc_aggregate(m_stack, ei, w):
    mesh = plsc.VectorSubcoreMesh(core_axis_name="c", subcore_axis_name="s")
    return pl.kernel(
        _sc_body,
        out_type=jax.ShapeDtypeStruct((Q * N, QC), jnp.float32),
        mesh=mesh,
        compiler_params=pltpu.CompilerParams(use_tc_tiling_on_sc=False),
        scratch_types=[
            pltpu.VMEM((CB,), jnp.int32),
            pltpu.VMEM((CB,), jnp.int32),
            pltpu.VMEM((2 * CBP + 16,), jnp.float32),
            pltpu.VMEM((NGRP, G), jnp.int32),
            pltpu.VMEM((NGRP, G), jnp.int32),
            pltpu.VMEM((G, QC), jnp.float32),
            pltpu.VMEM((G, QC), jnp.float32),
            pltpu.VMEM((G, QC), jnp.float32),
            pltpu.VMEM((G, QC), jnp.float32),
            pltpu.VMEM_SHARED((ACC_R, QC), jnp.float32),
            pltpu.SemaphoreType.DMA,
            pltpu.SemaphoreType.DMA,
            pltpu.SemaphoreType.DMA,
            pltpu.SemaphoreType.DMA,
            pltpu.SemaphoreType.DMA,
        ],
    )(m_stack, ei, w)


def _dense_kernel(s0_ref, s1_ref, s2_ref, s3_ref, x_ref, h0_ref, c0_ref,
                  w_ih_t_ref, w_hh_t_ref, b_ih_ref, b_hh_ref,
                  lw_ih_t_ref, lw_hh_t_ref, lb_ih_ref, lb_hh_ref,
                  lin_w_t_ref, lin_b_ref,
                  out_ref, h_new_ref, c_new_ref):
    x = x_ref[...]
    s3 = s3_ref[...]
    summed = jnp.concatenate(
        [s0_ref[...], s1_ref[...], s2_ref[...], s3[:, :CNT_LANE]], axis=1)
    agg = summed / jnp.clip(s3[:, CNT_LANE:CNT_LANE + 1], 1.0)
    gi = jnp.dot(agg, w_ih_t_ref[...], preferred_element_type=jnp.float32) + b_ih_ref[...]
    gh = jnp.dot(x, w_hh_t_ref[...], preferred_element_type=jnp.float32) + b_hh_ref[...]
    r = jax.nn.sigmoid(gi[:, :F] + gh[:, :F])
    z = jax.nn.sigmoid(gi[:, F:2 * F] + gh[:, F:2 * F])
    ng = jnp.tanh(gi[:, 2 * F:] + r * gh[:, 2 * F:])
    h = (1.0 - z) * ng + z * x
    gates = (jnp.dot(h, lw_ih_t_ref[...], preferred_element_type=jnp.float32)
             + lb_ih_ref[...]
             + jnp.dot(h0_ref[...], lw_hh_t_ref[...],
                       preferred_element_type=jnp.float32)
             + lb_hh_ref[...])
    i_g = jax.nn.sigmoid(gates[:, :F])
    f_g = jax.nn.sigmoid(gates[:, F:2 * F])
    g_g = jnp.tanh(gates[:, 2 * F:3 * F])
    o_g = jax.nn.sigmoid(gates[:, 3 * F:])
    c_new = f_g * c0_ref[...] + i_g * g_g
    h_new = o_g * jnp.tanh(c_new)
    out_ref[...] = (jnp.dot(jax.nn.relu(h_new), lin_w_t_ref[...],
                            preferred_element_type=jnp.float32)
                    + lin_b_ref[...])
    h_new_ref[...] = h_new
    c_new_ref[...] = c_new


def _dense_stage(summed_stack, x, h0, c0, gru_w_ih, gru_w_hh, gru_b_ih,
                 gru_b_hh, lstm_w_ih, lstm_w_hh, lstm_b_ih, lstm_b_hh,
                 lin_w, lin_b):
    T = lin_w.shape[0]
    nb = N // ROWS
    grid = (nb,)
    blk = lambda c: pl.BlockSpec((ROWS, c), lambda i: (i, 0))
    qblk = lambda q: pl.BlockSpec((ROWS, QC), lambda i, q=q: (q * nb + i, 0))
    full = lambda r, c: pl.BlockSpec((r, c), lambda i: (0, 0))
    return pl.pallas_call(
        _dense_kernel,
        grid=grid,
        in_specs=[
            qblk(0), qblk(1), qblk(2), qblk(3),
            blk(F), blk(F), blk(F),
            full(F, 3 * F), full(F, 3 * F), full(1, 3 * F), full(1, 3 * F),
            full(F, 4 * F), full(F, 4 * F), full(1, 4 * F), full(1, 4 * F),
            full(F, T), full(1, T),
        ],
        out_specs=[blk(T), blk(F), blk(F)],
        out_shape=[
            jax.ShapeDtypeStruct((N, T), jnp.float32),
            jax.ShapeDtypeStruct((N, F), jnp.float32),
            jax.ShapeDtypeStruct((N, F), jnp.float32),
        ],
    )(summed_stack, summed_stack, summed_stack, summed_stack,
      x, h0, c0,
      gru_w_ih.T, gru_w_hh.T, gru_b_ih.reshape(1, -1), gru_b_hh.reshape(1, -1),
      lstm_w_ih.T, lstm_w_hh.T, lstm_b_ih.reshape(1, -1),
      lstm_b_hh.reshape(1, -1), lin_w.T, lin_b.reshape(1, -1))


def kernel(x, edge_index, edge_weight, h_0, c_0, ggc_weight, gru_w_ih,
           gru_w_hh, gru_b_ih, gru_b_hh, lstm_w_ih, lstm_w_hh, lstm_b_ih,
           lstm_b_hh, lin_w, lin_b):
    w_pad = jnp.pad(ggc_weight[0], ((0, 0), (0, Q * QC - F)))
    w_quarters = w_pad.reshape(F, Q, QC).transpose(1, 0, 2)
    m_stack = _ggc_matmul(x, w_quarters)
    summed_stack = _sc_aggregate(m_stack, edge_index, edge_weight)
    out, h_new, c_new = _dense_stage(
        summed_stack, x, h_0[0], c_0[0], gru_w_ih, gru_w_hh, gru_b_ih,
        gru_b_hh, lstm_w_ih, lstm_w_hh, lstm_b_ih, lstm_b_hh, lin_w, lin_b)
    return (out, h_new[None], c_new[None])


# speedup vs baseline: 1.1288x; 1.1288x over previous
import jax
import jax.numpy as jnp
from jax import lax
from jax.experimental import pallas as pl
from jax.experimental.pallas import tpu as pltpu
from jax.experimental.pallas import tpu_sc as plsc

N = 50000
E = 800000
F = 100
Q = 4
QC = 32
CNT_Q = 3
CNT_LANE = 4

ROWS = 2000

NCORES = 2
NTILES = 16
EPT = E // NTILES
CB = 2000
CBP = 2048
NB = EPT // CB
G = 128
NGRP = CBP // G
ACC_R = 50048
DUMMY0 = N
ZSHARE = ACC_R // NTILES
ZROWS = 136
CO_SHARE = 3128
CO_LAST = N - 15 * CO_SHARE


def _matmul_kernel(x_ref, w_ref, o_ref):
    o_ref[...] = jnp.dot(x_ref[...], w_ref[0],
                         preferred_element_type=jnp.float32)


def _ggc_matmul(x, w):
    grid = (Q, N // ROWS)
    return pl.pallas_call(
        _matmul_kernel,
        grid=grid,
        in_specs=[
            pl.BlockSpec((ROWS, F), lambda q, i: (i, 0)),
            pl.BlockSpec((1, F, QC), lambda q, i: (q, 0, 0)),
        ],
        out_specs=pl.BlockSpec((ROWS, QC),
                               lambda q, i: (q * (N // ROWS) + i, 0)),
        out_shape=jax.ShapeDtypeStruct((Q * N, QC), jnp.float32),
    )(x, w)


def _sc_body(m_hbm, ei_hbm, w_hbm, out_hbm,
             src_v, dst_v, w_v, src2d, dst2d,
             rows_g0, rows_g1, rows_s0, rows_s1, acc,
             gs0, gs1, ss0, ss1, st):
    cid = lax.axis_index("c")
    tid = lax.axis_index("s")
    lane = lax.iota(jnp.int32, 16)
    zeros16 = jnp.zeros((16,), jnp.float32)

    for p in range(Q // NCORES):
        q = cid + NCORES * p
        qn = q * N

        for zr in range(G):
            rows_s0[zr, pl.ds(0, 16)] = zeros16
            rows_s0[zr, pl.ds(16, 16)] = zeros16

        def zero_step(j, _):
            pltpu.sync_copy(rows_s0, acc.at[pl.ds(tid * ZSHARE + j * G, G)])
            return 0
        lax.fori_loop(0, ZSHARE // G, zero_step, 0)
        pltpu.sync_copy(rows_s0.at[pl.ds(0, ZSHARE - (ZSHARE // G) * G)],
                        acc.at[pl.ds(tid * ZSHARE + (ZSHARE // G) * G,
                                     ZSHARE - (ZSHARE // G) * G)])

        dummy_src = qn + tid * 16 + lane
        for k in range(5, 8):
            src2d[15, pl.ds(k * 16, 16)] = dummy_src
            dst2d[15, pl.ds(k * 16, 16)] = DUMMY0 + lane
        for k in range(CB // 16, CBP // 16):
            w_v[pl.ds(k * 16, 16)] = zeros16
            w_v[pl.ds(CBP + k * 16, 16)] = zeros16

        def issue_stage(bk, whalf):
            ebase = tid * EPT + bk * CB
            pltpu.async_copy(ei_hbm.at[0, pl.ds(ebase, CB)], src_v, st)
            pltpu.async_copy(ei_hbm.at[1, pl.ds(ebase, CB)], dst_v, st)
            pltpu.async_copy(w_hbm.at[pl.ds(ebase, CB)],
                             w_v.at[pl.ds(whalf, CB)], st)

        issue_stage(0, 0)
        plsc.subcore_barrier()

        def block_step(bk, _):
            @pl.when(bk > 0)
            def _():
                pltpu.make_async_copy(rows_s0, acc.at[dst2d.at[0]],
                                      ss0).wait()
                pltpu.make_async_copy(rows_s1, acc.at[dst2d.at[1]],
                                      ss1).wait()

            half = lax.rem(bk, 2) * CBP
            ebase = tid * EPT + bk * CB
            pltpu.make_async_copy(ei_hbm.at[0, pl.ds(ebase, CB)],
                                  src_v, st).wait()
            pltpu.make_async_copy(ei_hbm.at[1, pl.ds(ebase, CB)],
                                  dst_v, st).wait()
            pltpu.make_async_copy(w_hbm.at[pl.ds(ebase, CB)],
                                  w_v.at[pl.ds(half, CB)], st).wait()

            for k in range(CB // 16):
                r, c = k // 8, (k % 8) * 16
                src2d[r, pl.ds(c, 16)] = src_v[pl.ds(k * 16, 16)] + qn
                dst2d[r, pl.ds(c, 16)] = dst_v[pl.ds(k * 16, 16)]

            @pl.when(bk < NB - 1)
            def _():
                issue_stage(bk + 1, CBP - half)

            indf = jnp.minimum(jnp.abs(lane - CNT_LANE), 1).astype(jnp.float32)
            cnt_add = cid.astype(jnp.float32) * (1.0 - indf)
            cnt_keep = 1.0 - cnt_add

            pltpu.async_copy(m_hbm.at[src2d.at[0]], rows_g0, gs0)
            pltpu.async_copy(m_hbm.at[src2d.at[1]], rows_g1, gs1)

            bufs = ((rows_g0, rows_s0, gs0, ss0), (rows_g1, rows_s1, gs1, ss1))

            def gpair(i, _):
                for bi, (rg, rs, gs, ss) in enumerate(bufs):
                    g = 2 * i + bi
                    pltpu.make_async_copy(m_hbm.at[src2d.at[g]], rg,
                                          gs).wait()

                    @pl.when(i >= 1)
                    def _():
                        pltpu.make_async_copy(rs, acc.at[dst2d.at[g]],
                                              ss).wait()

                    base = half + g * G
                    for j16 in range(G // 16):
                        w16 = w_v[pl.ds(base + j16 * 16, 16)]
                        for l in range(16):
                            j = j16 * 16 + l
                            wv = w16.at[jnp.full((16,), l, jnp.int32)].get(
                                mode="promise_in_bounds")
                            v0 = rg[j, pl.ds(0, 16)] * wv
                            if p == 1:
                                v0 = v0 * cnt_keep + cnt_add
                            rs[j, pl.ds(0, 16)] = v0
                            rs[j, pl.ds(16, 16)] = rg[j, pl.ds(16, 16)] * wv
                    pltpu.async_copy(rs, acc.at[dst2d.at[g]], ss, add=True)

                    @pl.when(i < NGRP // 2 - 1)
                    def _():
                        pltpu.async_copy(m_hbm.at[src2d.at[g + 2]], rg, gs)
                return 0

            lax.fori_loop(0, NGRP // 2, gpair, 0)
            return 0

        lax.fori_loop(0, NB, block_step, 0)
        pltpu.make_async_copy(rows_s0, acc.at[dst2d.at[0]], ss0).wait()
        pltpu.make_async_copy(rows_s1, acc.at[dst2d.at[1]], ss1).wait()
        plsc.subcore_barrier()

        @pl.when(tid < NTILES - 1)
        def _():
            pltpu.sync_copy(acc.at[pl.ds(tid * CO_SHARE, CO_SHARE)],
                            out_hbm.at[pl.ds(qn + tid * CO_SHARE, CO_SHARE)])

        @pl.when(tid == NTILES - 1)
        def _():
            pltpu.sync_copy(
                acc.at[pl.ds(15 * CO_SHARE, CO_LAST)],
                out_hbm.at[pl.ds(qn + 15 * CO_SHARE, CO_LAST)])
        plsc.subcore_barrier()


def _sc_aggregate(m_stack, ei, w):
    mesh = plsc.VectorSubcoreMesh(core_axis_name="c", subcore_axis_name="s")
    return pl.kernel(
        _sc_body,
        out_type=jax.ShapeDtypeStruct((Q * N, QC), jnp.float32),
        mesh=mesh,
        compiler_params=pltpu.CompilerParams(use_tc_tiling_on_sc=False),
        scratch_types=[
            pltpu.VMEM((CB,), jnp.int32),
            pltpu.VMEM((CB,), jnp.int32),
            pltpu.VMEM((2 * CBP + 16,), jnp.float32),
            pltpu.VMEM((NGRP, G), jnp.int32),
            pltpu.VMEM((NGRP, G), jnp.int32),
            pltpu.VMEM((G, QC), jnp.float32),
            pltpu.VMEM((G, QC), jnp.float32),
            pltpu.VMEM((G, QC), jnp.float32),
            pltpu.VMEM((G, QC), jnp.float32),
            pltpu.VMEM_SHARED((ACC_R, QC), jnp.float32),
            pltpu.SemaphoreType.DMA,
            pltpu.SemaphoreType.DMA,
            pltpu.SemaphoreType.DMA,
            pltpu.SemaphoreType.DMA,
            pltpu.SemaphoreType.DMA,
        ],
    )(m_stack, ei, w)


def _dense_kernel(s0_ref, s1_ref, s2_ref, s3_ref, x_ref, h0_ref, c0_ref,
                  w_ih_t_ref, w_hh_t_ref, b_ih_ref, b_hh_ref,
                  lw_ih_t_ref, lw_hh_t_ref, lb_ih_ref, lb_hh_ref,
                  lin_w_t_ref, lin_b_ref,
                  out_ref, h_new_ref, c_new_ref):
    x = x_ref[...]
    s3 = s3_ref[...]
    summed = jnp.concatenate(
        [s0_ref[...], s1_ref[...], s2_ref[...], s3[:, :CNT_LANE]], axis=1)
    agg = summed / jnp.clip(s3[:, CNT_LANE:CNT_LANE + 1], 1.0)
    gi = jnp.dot(agg, w_ih_t_ref[...], preferred_element_type=jnp.float32) + b_ih_ref[...]
    gh = jnp.dot(x, w_hh_t_ref[...], preferred_element_type=jnp.float32) + b_hh_ref[...]
    r = jax.nn.sigmoid(gi[:, :F] + gh[:, :F])
    z = jax.nn.sigmoid(gi[:, F:2 * F] + gh[:, F:2 * F])
    ng = jnp.tanh(gi[:, 2 * F:] + r * gh[:, 2 * F:])
    h = (1.0 - z) * ng + z * x
    gates = (jnp.dot(h, lw_ih_t_ref[...], preferred_element_type=jnp.float32)
             + lb_ih_ref[...]
             + jnp.dot(h0_ref[...], lw_hh_t_ref[...],
                       preferred_element_type=jnp.float32)
             + lb_hh_ref[...])
    i_g = jax.nn.sigmoid(gates[:, :F])
    f_g = jax.nn.sigmoid(gates[:, F:2 * F])
    g_g = jnp.tanh(gates[:, 2 * F:3 * F])
    o_g = jax.nn.sigmoid(gates[:, 3 * F:])
    c_new = f_g * c0_ref[...] + i_g * g_g
    h_new = o_g * jnp.tanh(c_new)
    out_ref[...] = (jnp.dot(jax.nn.relu(h_new), lin_w_t_ref[...],
                            preferred_element_type=jnp.float32)
                    + lin_b_ref[...])
    h_new_ref[...] = h_new
    c_new_ref[...] = c_new


def _dense_stage(summed_stack, x, h0, c0, gru_w_ih, gru_w_hh, gru_b_ih,
                 gru_b_hh, lstm_w_ih, lstm_w_hh, lstm_b_ih, lstm_b_hh,
                 lin_w, lin_b):
    T = lin_w.shape[0]
    nb = N // ROWS
    grid = (nb,)
    blk = lambda c: pl.BlockSpec((ROWS, c), lambda i: (i, 0))
    qblk = lambda q: pl.BlockSpec((ROWS, QC), lambda i, q=q: (q * nb + i, 0))
    full = lambda r, c: pl.BlockSpec((r, c), lambda i: (0, 0))
    return pl.pallas_call(
        _dense_kernel,
        grid=grid,
        in_specs=[
            qblk(0), qblk(1), qblk(2), qblk(3),
            blk(F), blk(F), blk(F),
            full(F, 3 * F), full(F, 3 * F), full(1, 3 * F), full(1, 3 * F),
            full(F, 4 * F), full(F, 4 * F), full(1, 4 * F), full(1, 4 * F),
            full(F, T), full(1, T),
        ],
        out_specs=[blk(T), blk(F), blk(F)],
        out_shape=[
            jax.ShapeDtypeStruct((N, T), jnp.float32),
            jax.ShapeDtypeStruct((N, F), jnp.float32),
            jax.ShapeDtypeStruct((N, F), jnp.float32),
        ],
    )(summed_stack, summed_stack, summed_stack, summed_stack,
      x, h0, c0,
      gru_w_ih.T, gru_w_hh.T, gru_b_ih.reshape(1, -1), gru_b_hh.reshape(1, -1),
      lstm_w_ih.T, lstm_w_hh.T, lstm_b_ih.reshape(1, -1),
      lstm_b_hh.reshape(1, -1), lin_w.T, lin_b.reshape(1, -1))


def kernel(x, edge_index, edge_weight, h_0, c_0, ggc_weight, gru_w_ih,
           gru_w_hh, gru_b_ih, gru_b_hh, lstm_w_ih, lstm_w_hh, lstm_b_ih,
           lstm_b_hh, lin_w, lin_b):
    w_pad = jnp.pad(ggc_weight[0], ((0, 0), (0, Q * QC - F)))
    w_quarters = w_pad.reshape(F, Q, QC).transpose(1, 0, 2)
    m_stack = _ggc_matmul(x, w_quarters)
    summed_stack = _sc_aggregate(m_stack, edge_index, edge_weight)
    out, h_new, c_new = _dense_stage(
        summed_stack, x, h_0[0], c_0[0], gru_w_ih, gru_w_hh, gru_b_ih,
        gru_b_hh, lstm_w_ih, lstm_w_hh, lstm_b_ih, lstm_b_hh, lin_w, lin_b)
    return (out, h_new[None], c_new[None])
